# Initial kernel scaffold; baseline (speedup 1.0000x reference)
import jax
import jax.numpy as jnp
from jax.experimental import pallas as pl


def kernel(x, ec1_w, ec1_b, ec2_w, ec2_b, ec3_w, ec3_b, ec4_w, ec4_b, ec5_w, ec5_b, qc_w, qc_b, emb, dc1_w, dc1_b, dc2_w, dc2_b, dc3_w, dc3_b, dc4_w, dc4_b, dc5_w, dc5_b):
    raise NotImplementedError("write your pallas kernel here")


# speedup vs baseline: 1.0305x; 1.0305x over previous
import jax
import jax.numpy as jnp
from jax.experimental import pallas as pl
from jax.experimental.pallas import tpu as pltpu

_F32 = jnp.float32
_PAD = 128
_TT = 512
_TLAT = 2048


def _lrelu(v):
    return jnp.where(v >= 0, v, 0.01 * v)


def _dot(a, b):
    return jnp.dot(a, b, preferred_element_type=_F32)


def _zero_guards(ref, t):
    c = ref.shape[0]
    ref[:, _PAD - 1:_PAD] = jnp.zeros((c, 1), _F32)
    ref[:, _PAD + t:_PAD + t + 1] = jnp.zeros((c, 1), _F32)


def _conv_s2_terms(n_in):
    terms = []
    for q in range(n_in // 2):
        row = []
        for dk in range(4):
            s = 2 * q + dk - 1
            row.append((dk, s % n_in, s // n_in))
        terms.append(row)
    return terms


def _convT_s2_terms(n_in):
    terms = []
    for a in range(n_in):
        terms.append([(1, a, 0), (3, (a - 1) % n_in, -1 if a == 0 else 0)])
        terms.append([(0, (a + 1) % n_in, 1 if a == n_in - 1 else 0), (2, a, 0)])
    order = []
    for a in range(n_in):
        order.append(terms[2 * a])
    out = [None] * (2 * n_in)
    for a in range(n_in):
        out[2 * a] = terms[2 * a]
        out[2 * a + 1] = terms[2 * a + 1]
    return out


_K3_TERMS = [[(0, 0, -1), (1, 0, 0), (2, 0, 1)]]
_K3T_TERMS = [[(0, 0, 1), (1, 0, 0), (2, 0, -1)]]


def _phase_layer(in_ref, out_ref, w, b, c_in, n_in, terms, act,
                 out_concat=False):
    c_out = w.shape[1]
    n_t = _TLAT // _TT

    def tile(j, _):
        t0 = j * _TT
        segs = []
        for p in range(n_in):
            sa = in_ref[p * c_in:(p + 1) * c_in, pl.ds(t0, _TT + 2 * _PAD)]
            segs.append(sa[:, _PAD - 1:_PAD + _TT + 1])
        outs = []
        for q, row in enumerate(terms):
            acc = None
            for (dk, p, sh) in row:
                term = _dot(w[dk], segs[p][:, 1 + sh:1 + sh + _TT])
                acc = term if acc is None else acc + term
            v = acc + b
            if act:
                v = _lrelu(v)
            if out_concat:
                outs.append(v)
            else:
                out_ref[q * c_out:(q + 1) * c_out, pl.ds(t0 + _PAD, _TT)] = v
        if out_concat:
            out_ref[:, pl.ds(t0, _TT)] = jnp.concatenate(outs, axis=0)
        return 0

    jax.lax.fori_loop(0, n_t, tile, 0)


def _enc_body(x_ref, w1, b1, w2, b2, w3, b3, w4, b4, w5, b5, wq, bq, h_ref,
              s0, s1, s2, s3, s4):
    s0[:, _PAD:_PAD + _TLAT] = x_ref[0]
    _zero_guards(s0, _TLAT)
    _phase_layer(s0, s1, w1[...], b1[...], 2, 8,
                 _conv_s2_terms(8), True)
    _zero_guards(s1, _TLAT)
    _phase_layer(s1, s2, w2[...], b2[...], 128, 4,
                 _conv_s2_terms(4), True)
    _zero_guards(s2, _TLAT)
    _phase_layer(s2, s3, w3[...], b3[...], 256, 2,
                 _conv_s2_terms(2), True)
    _zero_guards(s3, _TLAT)
    _phase_layer(s3, s4, w4[...], b4[...], 256, 1,
                 _K3_TERMS, True)

    w5v, b5v, wqv, bqv = w5[...], b5[...], wq[...], bq[...]

    def tile(j, _):
        seg = s4[:, pl.ds(j * _TT + _PAD, _TT)]
        h5 = _lrelu(_dot(w5v, seg) + b5v)
        h_ref[0, :, pl.ds(j * _TT, _TT)] = _dot(wqv, h5) + bqv
        return 0

    jax.lax.fori_loop(0, _TLAT // _TT, tile, 0)


def _vq_body(flat_ref, embt_ref, emb_ref, embsq_ref, q_ref, loss_ref):
    emb = emb_ref[...]
    embt = embt_ref[...]
    emb_sq = embsq_ref[...]
    n = flat_ref.shape[0]
    rt = 256

    def tile(j, ss):
        ft = flat_ref[pl.ds(j * rt, rt), :]
        scores = emb_sq - 2.0 * _dot(ft, embt)
        iota = jax.lax.broadcasted_iota(jnp.int32, scores.shape, 1)
        m = jnp.min(scores, axis=1, keepdims=True)
        idx = jnp.min(jnp.where(scores == m, iota, scores.shape[1]),
                      axis=1, keepdims=True)
        onehot = (iota == idx).astype(_F32)
        qt = _dot(onehot, emb)
        q_ref[pl.ds(j * rt, rt), :] = qt
        diff = qt - ft
        return ss + jnp.sum(diff * diff)

    ss = jax.lax.fori_loop(0, n // rt, tile, jnp.zeros((), _F32))
    loss_ref[...] = jnp.full((1, 1, 128), ss, _F32)


def _dec_body(q_ref, dw1, db1, dw2, db2, dw3, db3, dw4, db4, dw5, db5,
              out_ref, s1, s2, s3, s4):
    dw1v, db1v = dw1[...], db1[...]

    def tile(j, _):
        seg = q_ref[0, :, pl.ds(j * _TT, _TT)]
        s1[:, pl.ds(j * _TT + _PAD, _TT)] = _lrelu(_dot(dw1v, seg) + db1v)
        return 0

    jax.lax.fori_loop(0, _TLAT // _TT, tile, 0)
    _zero_guards(s1, _TLAT)
    _phase_layer(s1, s2, dw2[...], db2[...], 256, 1,
                 _K3T_TERMS, True)
    _zero_guards(s2, _TLAT)
    _phase_layer(s2, s3, dw3[...], db3[...], 256, 1,
                 _convT_s2_terms(1), True)
    _zero_guards(s3, _TLAT)
    _phase_layer(s3, s4, dw4[...], db4[...], 256, 2,
                 _convT_s2_terms(2), True)
    _zero_guards(s4, _TLAT)
    _phase_layer(s4, out_ref.at[0], dw5[...], db5[...], 128, 4,
                 _convT_s2_terms(4), False, out_concat=True)


def _full_spec(v):
    nd = v.ndim
    return pl.BlockSpec(v.shape, lambda i, _n=nd: (0,) * _n)


def _params():
    return pltpu.CompilerParams(
        dimension_semantics=("arbitrary",),
        vmem_limit_bytes=60 * 1024 * 1024,
    )


def _scratch(rows):
    return pltpu.VMEM((rows, _TLAT + 2 * _PAD), _F32)


def kernel(x, ec1_w, ec1_b, ec2_w, ec2_b, ec3_w, ec3_b, ec4_w, ec4_b,
           ec5_w, ec5_b, qc_w, qc_b, emb, dc1_w, dc1_b, dc2_w, dc2_b,
           dc3_w, dc3_b, dc4_w, dc4_b, dc5_w, dc5_b):
    b_sz, c_in, t_sz = x.shape
    t_lat = t_sz // 8

    xph = x.reshape(b_sz, 2, t_lat, 8).transpose(0, 3, 1, 2) \
           .reshape(b_sz, 16, t_lat)

    w1 = jnp.transpose(ec1_w, (2, 0, 1))
    w2 = jnp.transpose(ec2_w, (2, 0, 1))
    w3 = jnp.transpose(ec3_w, (2, 0, 1))
    w4 = jnp.transpose(ec4_w, (2, 0, 1))
    w5 = ec5_w[:, :, 0]
    wq = qc_w[:, :, 0]
    dw1 = dc1_w[:, :, 0].T
    dw2 = jnp.transpose(dc2_w, (2, 1, 0))
    dw3 = jnp.transpose(dc3_w, (2, 1, 0))
    dw4 = jnp.transpose(dc4_w, (2, 1, 0))
    dw5 = jnp.transpose(dc5_w, (2, 1, 0))

    col = lambda v: v.reshape(-1, 1)

    enc_ops = (xph, w1, col(ec1_b), w2, col(ec2_b), w3, col(ec3_b),
               w4, col(ec4_b), w5, col(ec5_b), wq, col(qc_b))
    h = pl.pallas_call(
        _enc_body,
        grid=(b_sz,),
        in_specs=[pl.BlockSpec((1, 16, t_lat), lambda i: (i, 0, 0))]
        + [_full_spec(v) for v in enc_ops[1:]],
        out_specs=pl.BlockSpec((1, 64, t_lat), lambda i: (i, 0, 0)),
        out_shape=jax.ShapeDtypeStruct((b_sz, 64, t_lat), _F32),
        scratch_shapes=[_scratch(16), _scratch(512), _scratch(512),
                        _scratch(256), _scratch(256)],
        compiler_params=_params(),
    )(*enc_ops)

    n_rows = b_sz * 64 * t_lat // 64
    flat = h.reshape(n_rows, 64)
    rows_blk = n_rows // b_sz
    embt = emb.T
    emb_sq = jnp.sum(emb * emb, axis=1)[None, :]
    qflat, losses = pl.pallas_call(
        _vq_body,
        grid=(b_sz,),
        in_specs=[pl.BlockSpec((rows_blk, 64), lambda i: (i, 0)),
                  _full_spec(embt), _full_spec(emb), _full_spec(emb_sq)],
        out_specs=(pl.BlockSpec((rows_blk, 64), lambda i: (i, 0)),
                   pl.BlockSpec((1, 1, 128), lambda i: (i, 0, 0))),
        out_shape=(jax.ShapeDtypeStruct((n_rows, 64), _F32),
                   jax.ShapeDtypeStruct((b_sz, 1, 128), _F32)),
        compiler_params=_params(),
    )(flat, embt, emb, emb_sq)

    q = qflat.reshape(b_sz, 64, t_lat)

    dec_ops = (q, dw1, col(dc1_b), dw2, col(dc2_b), dw3, col(dc3_b),
               dw4, col(dc4_b), dw5, col(dc5_b))
    dph = pl.pallas_call(
        _dec_body,
        grid=(b_sz,),
        in_specs=[pl.BlockSpec((1, 64, t_lat), lambda i: (i, 0, 0))]
        + [_full_spec(v) for v in dec_ops[1:]],
        out_specs=pl.BlockSpec((1, 16, t_lat), lambda i: (i, 0, 0)),
        out_shape=jax.ShapeDtypeStruct((b_sz, 16, t_lat), _F32),
        scratch_shapes=[_scratch(256), _scratch(256), _scratch(512),
                        _scratch(512)],
        compiler_params=_params(),
    )(*dec_ops)

    d = dph.reshape(b_sz, 8, 2, t_lat).transpose(0, 2, 3, 1) \
           .reshape(b_sz, 2, t_sz)
    latent_loss = 1.25 * jnp.sum(losses[:, 0, 0]) / (b_sz * 64 * t_lat)
    return (d, latent_loss)


# speedup vs baseline: 1.0328x; 1.0022x over previous
import jax
import jax.numpy as jnp
from jax.experimental import pallas as pl
from jax.experimental.pallas import tpu as pltpu

_F32 = jnp.float32
_PAD = 128
_TT = 512
_TLAT = 2048


def _lrelu(v):
    return jnp.where(v >= 0, v, 0.01 * v)


def _dot(a, b):
    return jnp.dot(a, b, preferred_element_type=_F32)


def _zero_guards(ref, t):
    c = ref.shape[0]
    ref[:, _PAD - 1:_PAD] = jnp.zeros((c, 1), _F32)
    ref[:, _PAD + t:_PAD + t + 1] = jnp.zeros((c, 1), _F32)


def _conv_s2_terms(n_in):
    terms = []
    for q in range(n_in // 2):
        row = []
        for dk in range(4):
            s = 2 * q + dk - 1
            row.append((dk, s % n_in, s // n_in))
        terms.append(row)
    return terms


def _convT_s2_terms(n_in):
    terms = []
    for a in range(n_in):
        terms.append([(1, a, 0), (3, (a - 1) % n_in, -1 if a == 0 else 0)])
        terms.append([(0, (a + 1) % n_in, 1 if a == n_in - 1 else 0), (2, a, 0)])
    order = []
    for a in range(n_in):
        order.append(terms[2 * a])
    out = [None] * (2 * n_in)
    for a in range(n_in):
        out[2 * a] = terms[2 * a]
        out[2 * a + 1] = terms[2 * a + 1]
    return out


_K3_TERMS = [[(0, 0, -1), (1, 0, 0), (2, 0, 1)]]
_K3T_TERMS = [[(0, 0, 1), (1, 0, 0), (2, 0, -1)]]


def _phase_layer(in_ref, out_ref, w, b, c_in, n_in, terms, act,
                 out_concat=False):
    c_out = w.shape[1]
    n_t = _TLAT // _TT

    def tile(j, _):
        t0 = j * _TT
        segs = []
        for p in range(n_in):
            sa = in_ref[p * c_in:(p + 1) * c_in, pl.ds(t0, _TT + 2 * _PAD)]
            segs.append(sa[:, _PAD - 1:_PAD + _TT + 1])
        outs = []
        for q, row in enumerate(terms):
            acc = None
            for (dk, p, sh) in row:
                term = _dot(w[dk], segs[p][:, 1 + sh:1 + sh + _TT])
                acc = term if acc is None else acc + term
            v = acc + b
            if act:
                v = _lrelu(v)
            if out_concat:
                outs.append(v)
            else:
                out_ref[q * c_out:(q + 1) * c_out, pl.ds(t0 + _PAD, _TT)] = v
        if out_concat:
            out_ref[:, pl.ds(t0, _TT)] = jnp.concatenate(outs, axis=0)
        return 0

    jax.lax.fori_loop(0, n_t, tile, 0)


def _enc_body(x_ref, w1, b1, w2, b2, w3, b3, w4, b4, w5, b5, wq, bq, h_ref,
              s0, s1, s2, s3, s4):
    s0[:, _PAD:_PAD + _TLAT] = x_ref[0]
    _zero_guards(s0, _TLAT)
    _phase_layer(s0, s1, w1[...], b1[...], 2, 8,
                 _conv_s2_terms(8), True)
    _zero_guards(s1, _TLAT)
    _phase_layer(s1, s2, w2[...], b2[...], 128, 4,
                 _conv_s2_terms(4), True)
    _zero_guards(s2, _TLAT)
    _phase_layer(s2, s3, w3[...], b3[...], 256, 2,
                 _conv_s2_terms(2), True)
    _zero_guards(s3, _TLAT)
    _phase_layer(s3, s4, w4[...], b4[...], 256, 1,
                 _K3_TERMS, True)

    w5v, b5v, wqv, bqv = w5[...], b5[...], wq[...], bq[...]

    def tile(j, _):
        seg = s4[:, pl.ds(j * _TT + _PAD, _TT)]
        h5 = _lrelu(_dot(w5v, seg) + b5v)
        h_ref[0, :, pl.ds(j * _TT, _TT)] = _dot(wqv, h5) + bqv
        return 0

    jax.lax.fori_loop(0, _TLAT // _TT, tile, 0)


def _vq_body(flat_ref, embt_ref, emb_ref, embsq_ref, q_ref, loss_ref):
    emb = emb_ref[...]
    embt = embt_ref[...]
    emb_sq = embsq_ref[...]
    n = flat_ref.shape[0]
    rt = 256

    def tile(j, ss):
        ft = flat_ref[pl.ds(j * rt, rt), :]
        scores = emb_sq - 2.0 * _dot(ft, embt)
        iota = jax.lax.broadcasted_iota(jnp.int32, scores.shape, 1)
        m = jnp.min(scores, axis=1, keepdims=True)
        idx = jnp.min(jnp.where(scores == m, iota, scores.shape[1]),
                      axis=1, keepdims=True)
        onehot = (iota == idx).astype(_F32)
        qt = _dot(onehot, emb)
        q_ref[pl.ds(j * rt, rt), :] = qt
        diff = qt - ft
        return ss + jnp.sum(diff * diff)

    ss = jax.lax.fori_loop(0, n // rt, tile, jnp.zeros((), _F32))
    loss_ref[...] = jnp.full((1, 1, 128), ss, _F32)


def _dec_body(q_ref, dw1, db1, dw2, db2, dw3, db3, dw4, db4, dw5, db5,
              out_ref, s1, s2, s3, s4):
    dw1v, db1v = dw1[...], db1[...]

    def tile(j, _):
        seg = q_ref[0, :, pl.ds(j * _TT, _TT)]
        s1[:, pl.ds(j * _TT + _PAD, _TT)] = _lrelu(_dot(dw1v, seg) + db1v)
        return 0

    jax.lax.fori_loop(0, _TLAT // _TT, tile, 0)
    _zero_guards(s1, _TLAT)
    _phase_layer(s1, s2, dw2[...], db2[...], 256, 1,
                 _K3T_TERMS, True)
    _zero_guards(s2, _TLAT)
    _phase_layer(s2, s3, dw3[...], db3[...], 256, 1,
                 _convT_s2_terms(1), True)
    _zero_guards(s3, _TLAT)
    _phase_layer(s3, s4, dw4[...], db4[...], 256, 2,
                 _convT_s2_terms(2), True)
    _zero_guards(s4, _TLAT)
    _phase_layer(s4, out_ref.at[0], dw5[...], db5[...], 128, 4,
                 _convT_s2_terms(4), False, out_concat=True)


def _full_spec(v):
    nd = v.ndim
    return pl.BlockSpec(v.shape, lambda i, _n=nd: (0,) * _n)


def _params():
    return pltpu.CompilerParams(
        dimension_semantics=("parallel",),
        vmem_limit_bytes=60 * 1024 * 1024,
    )


def _scratch(rows):
    return pltpu.VMEM((rows, _TLAT + 2 * _PAD), _F32)


def kernel(x, ec1_w, ec1_b, ec2_w, ec2_b, ec3_w, ec3_b, ec4_w, ec4_b,
           ec5_w, ec5_b, qc_w, qc_b, emb, dc1_w, dc1_b, dc2_w, dc2_b,
           dc3_w, dc3_b, dc4_w, dc4_b, dc5_w, dc5_b):
    b_sz, c_in, t_sz = x.shape
    t_lat = t_sz // 8

    xph = x.reshape(b_sz, 2, t_lat, 8).transpose(0, 3, 1, 2) \
           .reshape(b_sz, 16, t_lat)

    w1 = jnp.transpose(ec1_w, (2, 0, 1))
    w2 = jnp.transpose(ec2_w, (2, 0, 1))
    w3 = jnp.transpose(ec3_w, (2, 0, 1))
    w4 = jnp.transpose(ec4_w, (2, 0, 1))
    w5 = ec5_w[:, :, 0]
    wq = qc_w[:, :, 0]
    dw1 = dc1_w[:, :, 0].T
    dw2 = jnp.transpose(dc2_w, (2, 1, 0))
    dw3 = jnp.transpose(dc3_w, (2, 1, 0))
    dw4 = jnp.transpose(dc4_w, (2, 1, 0))
    dw5 = jnp.transpose(dc5_w, (2, 1, 0))

    col = lambda v: v.reshape(-1, 1)

    enc_ops = (xph, w1, col(ec1_b), w2, col(ec2_b), w3, col(ec3_b),
               w4, col(ec4_b), w5, col(ec5_b), wq, col(qc_b))
    h = pl.pallas_call(
        _enc_body,
        grid=(b_sz,),
        in_specs=[pl.BlockSpec((1, 16, t_lat), lambda i: (i, 0, 0))]
        + [_full_spec(v) for v in enc_ops[1:]],
        out_specs=pl.BlockSpec((1, 64, t_lat), lambda i: (i, 0, 0)),
        out_shape=jax.ShapeDtypeStruct((b_sz, 64, t_lat), _F32),
        scratch_shapes=[_scratch(16), _scratch(512), _scratch(512),
                        _scratch(256), _scratch(256)],
        compiler_params=_params(),
    )(*enc_ops)

    n_rows = b_sz * 64 * t_lat // 64
    flat = h.reshape(n_rows, 64)
    rows_blk = n_rows // b_sz
    embt = emb.T
    emb_sq = jnp.sum(emb * emb, axis=1)[None, :]
    qflat, losses = pl.pallas_call(
        _vq_body,
        grid=(b_sz,),
        in_specs=[pl.BlockSpec((rows_blk, 64), lambda i: (i, 0)),
                  _full_spec(embt), _full_spec(emb), _full_spec(emb_sq)],
        out_specs=(pl.BlockSpec((rows_blk, 64), lambda i: (i, 0)),
                   pl.BlockSpec((1, 1, 128), lambda i: (i, 0, 0))),
        out_shape=(jax.ShapeDtypeStruct((n_rows, 64), _F32),
                   jax.ShapeDtypeStruct((b_sz, 1, 128), _F32)),
        compiler_params=_params(),
    )(flat, embt, emb, emb_sq)

    q = qflat.reshape(b_sz, 64, t_lat)

    dec_ops = (q, dw1, col(dc1_b), dw2, col(dc2_b), dw3, col(dc3_b),
               dw4, col(dc4_b), dw5, col(dc5_b))
    dph = pl.pallas_call(
        _dec_body,
        grid=(b_sz,),
        in_specs=[pl.BlockSpec((1, 64, t_lat), lambda i: (i, 0, 0))]
        + [_full_spec(v) for v in dec_ops[1:]],
        out_specs=pl.BlockSpec((1, 16, t_lat), lambda i: (i, 0, 0)),
        out_shape=jax.ShapeDtypeStruct((b_sz, 16, t_lat), _F32),
        scratch_shapes=[_scratch(256), _scratch(256), _scratch(512),
                        _scratch(512)],
        compiler_params=_params(),
    )(*dec_ops)

    d = dph.reshape(b_sz, 8, 2, t_lat).transpose(0, 2, 3, 1) \
           .reshape(b_sz, 2, t_sz)
    latent_loss = 1.25 * jnp.sum(losses[:, 0, 0]) / (b_sz * 64 * t_lat)
    return (d, latent_loss)
